# Initial kernel scaffold; baseline (speedup 1.0000x reference)
import jax
import jax.numpy as jnp
from jax.experimental import pallas as pl


def kernel(uv, Z):
    raise NotImplementedError("write your pallas kernel here")


# speedup vs baseline: 1.1660x; 1.1660x over previous
import functools

import jax
import jax.numpy as jnp
import numpy as np
from jax import lax
from jax.experimental import pallas as pl
from jax.experimental.pallas import tpu as pltpu
from jax.experimental.pallas import tpu_sc as plsc

H = 2048
W = 2048
C = 16
B = 1048576

NC = 2
NS = 16
L = 16
NW = NC * NS
BPW = B // NW
CHUNK = 512
NGRP = CHUNK // L
NCHUNK = BPW // CHUNK

def _sc_bilinear(ux_hbm, uy_hbm, tex_hbm, out_hbm,
                 ux_v, uy_v, idx_v, w_v, t00_v, t01_v, t10_v, t11_v, out_v,
                 sem):
    wid = lax.axis_index("s") * NC + lax.axis_index("c")
    qbase = wid * BPW

    def chunk_body(ci, carry):
        start = qbase + ci * CHUNK
        pltpu.sync_copy(ux_hbm.at[pl.ds(start, CHUNK)], ux_v)
        pltpu.sync_copy(uy_hbm.at[pl.ds(start, CHUNK)], uy_v)

        def idx_body(g, carry2):
            off = g * L
            xs = ux_v[pl.ds(off, L)]
            ys = uy_v[pl.ds(off, L)]
            fx = ((xs * 2.0 - 1.0) + 1.0) * W
            fx = (fx - 1.0) * 0.5
            fy = ((ys * 2.0 - 1.0) + 1.0) * H
            fy = (fy - 1.0) * 0.5
            fx = jnp.clip(fx, 0.0, float(W - 1))
            fy = jnp.clip(fy, 0.0, float(H - 1))
            ix0 = fx.astype(jnp.int32)
            iy0 = fy.astype(jnp.int32)
            wx1 = fx - ix0.astype(jnp.float32)
            wy1 = fy - iy0.astype(jnp.float32)
            wx0 = 1.0 - wx1
            wy0 = 1.0 - wy1
            ix1 = jnp.minimum(ix0 + 1, W - 1)
            iy1 = jnp.minimum(iy0 + 1, H - 1)
            r0 = iy0 * W
            r1 = iy1 * W
            idx_v[0, pl.ds(off, L)] = r0 + ix0
            idx_v[1, pl.ds(off, L)] = r0 + ix1
            idx_v[2, pl.ds(off, L)] = r1 + ix0
            idx_v[3, pl.ds(off, L)] = r1 + ix1
            w_v[0, pl.ds(off, L)] = wy0 * wx0
            w_v[1, pl.ds(off, L)] = wy0 * wx1
            w_v[2, pl.ds(off, L)] = wy1 * wx0
            w_v[3, pl.ds(off, L)] = wy1 * wx1
            return carry2

        lax.fori_loop(0, NGRP, idx_body, 0, unroll=2)

        c0 = pltpu.async_copy(tex_hbm.at[idx_v.at[0]], t00_v, sem)
        c1 = pltpu.async_copy(tex_hbm.at[idx_v.at[1]], t01_v, sem)
        c2 = pltpu.async_copy(tex_hbm.at[idx_v.at[2]], t10_v, sem)
        c3 = pltpu.async_copy(tex_hbm.at[idx_v.at[3]], t11_v, sem)
        c0.wait()
        c1.wait()
        c2.wait()
        c3.wait()

        def grp_body(g, carry2):
            off = g * L
            zero = lax.iota(jnp.int32, L) * 0
            w00 = w_v[0, pl.ds(off, L)]
            w01 = w_v[1, pl.ds(off, L)]
            w10 = w_v[2, pl.ds(off, L)]
            w11 = w_v[3, pl.ds(off, L)]
            for lane in range(L):
                q = off + lane
                lv = zero + lane
                acc = (w00.at[lv].get(mode="promise_in_bounds") * t00_v[q]
                       + w01.at[lv].get(mode="promise_in_bounds") * t01_v[q]
                       + w10.at[lv].get(mode="promise_in_bounds") * t10_v[q]
                       + w11.at[lv].get(mode="promise_in_bounds") * t11_v[q])
                out_v[q] = acc
            return carry2

        lax.fori_loop(0, NGRP, grp_body, 0)

        pltpu.sync_copy(out_v, out_hbm.at[pl.ds(start, CHUNK)])
        return carry

    lax.fori_loop(0, NCHUNK, chunk_body, 0)


@functools.partial(
    pl.kernel,
    mesh=plsc.VectorSubcoreMesh(core_axis_name="c", subcore_axis_name="s"),
    out_type=jax.ShapeDtypeStruct((B, C), jnp.float32),
    compiler_params=pltpu.CompilerParams(use_tc_tiling_on_sc=False),
    scratch_types=[
        pltpu.VMEM((CHUNK,), jnp.float32),
        pltpu.VMEM((CHUNK,), jnp.float32),
        pltpu.VMEM((4, CHUNK), jnp.int32),
        pltpu.VMEM((4, CHUNK), jnp.float32),
        pltpu.VMEM((CHUNK, C), jnp.float32),
        pltpu.VMEM((CHUNK, C), jnp.float32),
        pltpu.VMEM((CHUNK, C), jnp.float32),
        pltpu.VMEM((CHUNK, C), jnp.float32),
        pltpu.VMEM((CHUNK, C), jnp.float32),
        pltpu.SemaphoreType.DMA,
    ],
)
def _sc_kernel(ux_hbm, uy_hbm, tex_hbm, out_hbm, *scratch):
    _sc_bilinear(ux_hbm, uy_hbm, tex_hbm, out_hbm, *scratch)


def kernel(uv, Z):
    tex = jnp.transpose(Z[0], (1, 2, 0)).reshape(H * W, C)
    ux = uv[:, 0]
    uy = uv[:, 1]
    return _sc_kernel(ux, uy, tex)
